# Initial kernel scaffold; baseline (speedup 1.0000x reference)
import jax
import jax.numpy as jnp
from jax.experimental import pallas as pl


def kernel(x, wq, gq, bq, wk, gk, bk, wv, lw, lb, ow, ob):
    raise NotImplementedError("write your pallas kernel here")


# speedup vs baseline: 1.5660x; 1.5660x over previous
import functools

import jax
import jax.numpy as jnp
from jax.experimental import pallas as pl
import jax.experimental.pallas.tpu as pltpu

DIM = 96
NH = 8
HD = DIM // NH
TOPK = 4
THETA = 0.2
SCALE = DIM ** (-0.5)
EPS = 1e-5

H = W = 56
HP = H * W
PADR = 64
SROWS = PADR + HP + PADR
RS = 196
RSP = 208
NREG_HW = 16

f32 = jnp.float32
bf16 = jnp.bfloat16


def _wcol_masks():
    wcol = jax.lax.broadcasted_iota(jnp.int32, (HP, 1), 0) % W
    m_m1 = (wcol >= 1).astype(f32)
    m_p1 = (wcol <= W - 2).astype(f32)
    return m_m1, m_p1


def _conv_kernel(xp_ref, xc_ref, xn_ref, wtap_ref, kd_ref, wv_ref,
                 qk_ref, v_ref, pool_ref, scr):
    t = pl.program_id(1)
    T = pl.num_programs(1)
    scr[:, :PADR, :] = jnp.zeros((3, PADR, DIM), f32)
    scr[:, PADR + HP:, :] = jnp.zeros((3, PADR, DIM), f32)
    mp = (t > 0).astype(f32)
    mn = (t < T - 1).astype(f32)
    scr[0, PADR:PADR + HP, :] = xp_ref[0, 0] * mp
    scr[1, PADR:PADR + HP, :] = xc_ref[0, 0]
    scr[2, PADR:PADR + HP, :] = xn_ref[0, 0] * mn

    m_m1, m_p1 = _wcol_masks()
    wmask = {-1: m_m1, 1: m_p1}

    acc = jnp.zeros((HP, 2 * DIM), f32)
    i = 0
    for dt in range(3):
        for a in (-1, 0, 1):
            for b in (-1, 0, 1):
                off = PADR + a * W + b
                win = scr[dt, off:off + HP, :]
                if b != 0:
                    win = win * wmask[b]
                acc = acc + jnp.dot(win.astype(bf16), wtap_ref[i],
                                    preferred_element_type=f32)
                i += 1

    xc16 = xc_ref[0, 0].astype(bf16)
    acc = acc - THETA * jnp.dot(xc16, kd_ref[...], preferred_element_type=f32)

    v = jnp.dot(xc16, wv_ref[...], preferred_element_type=f32)

    ir = jax.lax.broadcasted_iota(jnp.int32, (NREG_HW + 1, HP), 0)
    p = jax.lax.broadcasted_iota(jnp.int32, (NREG_HW + 1, HP), 1)
    reg = (p // (W * 14)) * 4 + (p % W) // 14
    pmat = ((ir == reg) | (ir == NREG_HW)).astype(bf16)
    acc_hi = acc.astype(bf16)
    acc_lo = (acc - acc_hi.astype(f32)).astype(bf16)
    nt = (((1,), (0,)), ((), ()))
    pooled = (jax.lax.dot_general(pmat, acc_hi, nt, preferred_element_type=f32)
              + jax.lax.dot_general(pmat, acc_lo, nt, preferred_element_type=f32))
    sq = jnp.sum(acc * acc, axis=0).reshape(1, 2 * DIM)

    qk_ref[0, 0] = acc
    v_ref[0, 0] = v
    pool_ref[0, 0, :NREG_HW + 1, :] = pooled
    pool_ref[0, 0, NREG_HW + 1:, :] = sq


def _topk_kernel(rs_ref, scale_ref, shift_ref, idx_ref):
    rs = rs_ref[0] * (1.0 / RS)
    qr = rs[:, :DIM] * scale_ref[0, :DIM] + shift_ref[0, :DIM]
    kr = rs[:, DIM:] * scale_ref[0, DIM:] + shift_ref[0, DIM:]
    a = jax.lax.dot_general(qr.astype(bf16), kr.astype(bf16),
                            (((1,), (1,)), ((), ())),
                            preferred_element_type=f32)
    n = a.shape[0]
    iota = jax.lax.broadcasted_iota(jnp.int32, (n, n), 1)
    cols = []
    for _ in range(TOPK):
        m = jnp.max(a, axis=1, keepdims=True)
        col = jnp.min(jnp.where(a >= m, iota, n), axis=1)
        cols.append(col.reshape(n, 1))
        a = jnp.where(iota == col[:, None], -jnp.inf, a)
    idx_ref[0] = jnp.concatenate(cols, axis=1)


def _attn_kernel(idx_ref, q_ref, k0_ref, k1_ref, k2_ref, k3_ref,
                 v0_ref, v1_ref, v2_ref, v3_ref,
                 qsc_ref, qsh_ref, ksc_ref, ksh_ref, o_ref):
    del idx_ref
    qb = (q_ref[0, 0] * qsc_ref[0] + qsh_ref[0]) * SCALE
    kb = jnp.concatenate([k0_ref[0, 0], k1_ref[0, 0],
                          k2_ref[0, 0], k3_ref[0, 0]], axis=0)
    kb = kb * ksc_ref[0] + ksh_ref[0]
    vb = jnp.concatenate([v0_ref[0, 0], v1_ref[0, 0],
                          v2_ref[0, 0], v3_ref[0, 0]], axis=0)
    q16, k16, v16 = qb.astype(bf16), kb.astype(bf16), vb.astype(bf16)
    nk = TOPK * RSP
    kmask = (jax.lax.broadcasted_iota(jnp.int32, (1, nk), 1) % RSP) < RS
    outs = []
    for h in range(NH):
        sl = slice(h * HD, (h + 1) * HD)
        s = jax.lax.dot_general(q16[:, sl], k16[:, sl],
                                (((1,), (1,)), ((), ())),
                                preferred_element_type=f32)
        s = jnp.where(kmask, s, -jnp.inf)
        m = jnp.max(s, axis=1, keepdims=True)
        e = jnp.exp(s - m)
        p = e / jnp.sum(e, axis=1, keepdims=True)
        outs.append(jnp.dot(p.astype(bf16), v16[:, sl],
                            preferred_element_type=f32))
    o_ref[0, 0] = jnp.concatenate(outs, axis=1)


def _lepe_kernel(at_ref, vp_ref, vc_ref, vn_ref, lwt_ref, owt_ref,
                 lb_ref, ob_ref, o_ref, scr):
    t = pl.program_id(1)
    T = pl.num_programs(1)
    scr[:, :PADR, :] = jnp.zeros((3, PADR, DIM), f32)
    scr[:, PADR + HP:, :] = jnp.zeros((3, PADR, DIM), f32)
    mp = (t > 0).astype(f32)
    mn = (t < T - 1).astype(f32)
    scr[0, PADR:PADR + HP, :] = vp_ref[0, 0] * mp
    scr[1, PADR:PADR + HP, :] = vc_ref[0, 0]
    scr[2, PADR:PADR + HP, :] = vn_ref[0, 0] * mn

    m_m1, m_p1 = _wcol_masks()
    wmask = {-1: m_m1, 1: m_p1}

    lw32 = lwt_ref[...].astype(bf16).astype(f32)
    lepe = jnp.zeros((HP, DIM), f32)
    i = 0
    for dt in range(3):
        for a in (-1, 0, 1):
            for b in (-1, 0, 1):
                off = PADR + a * W + b
                win = scr[dt, off:off + HP, :].astype(bf16).astype(f32)
                term = win * lw32[i].reshape(1, DIM)
                if b != 0:
                    term = term * wmask[b]
                lepe = lepe + term
                i += 1

    total = at_ref[0, 0] + lepe + lb_ref[0]
    out = jnp.dot(total.astype(bf16), owt_ref[...],
                  preferred_element_type=f32) + ob_ref[0]
    o_ref[0, 0] = out


def _tap_weights(w):
    o, i = w.shape[0], w.shape[1]
    return w.transpose(2, 3, 4, 1, 0).reshape(27, i, o)


def _cdc_diff_weight(w):
    kd = w[:, :, 0].sum(axis=(2, 3)) + w[:, :, 2].sum(axis=(2, 3))
    return kd.T


@jax.jit
def kernel(x, wq, gq, bq, wk, gk, bk, wv, lw, lb, ow, ob):
    B, C, T, _, _ = x.shape
    R = T * NREG_HW
    N = B * T * H * W

    xf = x.transpose(0, 2, 3, 4, 1).reshape(B, T, HP, C)
    wqk = jnp.concatenate([_tap_weights(wq), _tap_weights(wk)],
                          axis=2).astype(bf16)
    kd2 = jnp.concatenate([_cdc_diff_weight(wq), _cdc_diff_weight(wk)],
                          axis=1).astype(bf16)
    wv_t = wv.T.astype(bf16)

    grid1 = (B, T)
    qk_f, v_f, pools = pl.pallas_call(
        _conv_kernel,
        grid=grid1,
        in_specs=[
            pl.BlockSpec((1, 1, HP, C),
                         lambda b, t: (b, jnp.maximum(t - 1, 0), 0, 0)),
            pl.BlockSpec((1, 1, HP, C), lambda b, t: (b, t, 0, 0)),
            pl.BlockSpec((1, 1, HP, C),
                         lambda b, t: (b, jnp.minimum(t + 1, T - 1), 0, 0)),
            pl.BlockSpec((27, C, 2 * C), lambda b, t: (0, 0, 0)),
            pl.BlockSpec((C, 2 * C), lambda b, t: (0, 0)),
            pl.BlockSpec((C, C), lambda b, t: (0, 0)),
        ],
        out_specs=[
            pl.BlockSpec((1, 1, HP, 2 * C), lambda b, t: (b, t, 0, 0)),
            pl.BlockSpec((1, 1, HP, C), lambda b, t: (b, t, 0, 0)),
            pl.BlockSpec((1, 1, NREG_HW + 2, 2 * C),
                         lambda b, t: (b, t, 0, 0)),
        ],
        out_shape=[
            jax.ShapeDtypeStruct((B, T, HP, 2 * C), f32),
            jax.ShapeDtypeStruct((B, T, HP, C), f32),
            jax.ShapeDtypeStruct((B, T, NREG_HW + 2, 2 * C), f32),
        ],
        scratch_shapes=[pltpu.VMEM((3, SROWS, DIM), f32)],
        compiler_params=pltpu.CompilerParams(
            dimension_semantics=("parallel", "arbitrary")),
    )(xf, xf, xf, wqk, kd2, wv_t)

    ssum = pools[:, :, NREG_HW, :].sum(axis=(0, 1))
    ssq = pools[:, :, NREG_HW + 1, :].sum(axis=(0, 1))
    mean = ssum / N
    var = ssq / N - mean * mean
    g2 = jnp.concatenate([gq, gk])
    b2 = jnp.concatenate([bq, bk])
    scale = g2 / jnp.sqrt(var + EPS)
    shift = b2 - mean * scale
    scale2 = scale.reshape(1, 2 * C)
    shift2 = shift.reshape(1, 2 * C)

    rsums = pools[:, :, :NREG_HW, :].reshape(B, R, 2 * C)
    idx = pl.pallas_call(
        _topk_kernel,
        grid=(B,),
        in_specs=[
            pl.BlockSpec((1, R, 2 * C), lambda b: (b, 0, 0)),
            pl.BlockSpec((1, 2 * C), lambda b: (0, 0)),
            pl.BlockSpec((1, 2 * C), lambda b: (0, 0)),
        ],
        out_specs=pl.BlockSpec((1, R, TOPK), lambda b: (b, 0, 0)),
        out_shape=jax.ShapeDtypeStruct((B, R, TOPK), jnp.int32),
    )(rsums, scale2, shift2)

    def to_regions(a):
        c = a.shape[-1]
        a = a.reshape(B, T, 4, 14, 4, 14, c).transpose(0, 1, 2, 4, 3, 5, 6)
        a = a.reshape(B, R, RS, c)
        return jnp.pad(a, ((0, 0), (0, 0), (0, RSP - RS), (0, 0)))

    qreg = to_regions(qk_f[:, :, :, :C])
    kreg = to_regions(qk_f[:, :, :, C:])
    vreg = to_regions(v_f)

    qsc, qsh = scale2[:, :C], shift2[:, :C]
    ksc, ksh = scale2[:, C:], shift2[:, C:]

    def qmap(b, r, idx_ref):
        return (b, r, 0, 0)

    def gmap(j):
        def m(b, r, idx_ref):
            return (b, idx_ref[b, r, j], 0, 0)
        return m

    def vecmap(b, r, idx_ref):
        return (0, 0)

    gspec = pltpu.PrefetchScalarGridSpec(
        num_scalar_prefetch=1,
        grid=(B, R),
        in_specs=[
            pl.BlockSpec((1, 1, RSP, C), qmap),
            pl.BlockSpec((1, 1, RSP, C), gmap(0)),
            pl.BlockSpec((1, 1, RSP, C), gmap(1)),
            pl.BlockSpec((1, 1, RSP, C), gmap(2)),
            pl.BlockSpec((1, 1, RSP, C), gmap(3)),
            pl.BlockSpec((1, 1, RSP, C), gmap(0)),
            pl.BlockSpec((1, 1, RSP, C), gmap(1)),
            pl.BlockSpec((1, 1, RSP, C), gmap(2)),
            pl.BlockSpec((1, 1, RSP, C), gmap(3)),
            pl.BlockSpec((1, C), vecmap),
            pl.BlockSpec((1, C), vecmap),
            pl.BlockSpec((1, C), vecmap),
            pl.BlockSpec((1, C), vecmap),
        ],
        out_specs=pl.BlockSpec((1, 1, RSP, C), qmap),
    )
    attn_out = pl.pallas_call(
        _attn_kernel,
        grid_spec=gspec,
        out_shape=jax.ShapeDtypeStruct((B, R, RSP, C), f32),
        compiler_params=pltpu.CompilerParams(
            dimension_semantics=("parallel", "arbitrary")),
    )(idx, qreg, kreg, kreg, kreg, kreg, vreg, vreg, vreg, vreg,
      qsc, qsh, ksc, ksh)

    at = attn_out[:, :, :RS, :].reshape(B, T, 4, 4, 14, 14, C)
    at = at.transpose(0, 1, 2, 4, 3, 5, 6).reshape(B, T, HP, C)

    lwt = lw[:, 0].transpose(1, 2, 3, 0).reshape(27, C)
    ow_t = ow.T.astype(bf16)
    lb2 = lb.reshape(1, C)
    ob2 = ob.reshape(1, C)

    out_f = pl.pallas_call(
        _lepe_kernel,
        grid=grid1,
        in_specs=[
            pl.BlockSpec((1, 1, HP, C), lambda b, t: (b, t, 0, 0)),
            pl.BlockSpec((1, 1, HP, C),
                         lambda b, t: (b, jnp.maximum(t - 1, 0), 0, 0)),
            pl.BlockSpec((1, 1, HP, C), lambda b, t: (b, t, 0, 0)),
            pl.BlockSpec((1, 1, HP, C),
                         lambda b, t: (b, jnp.minimum(t + 1, T - 1), 0, 0)),
            pl.BlockSpec((27, C), lambda b, t: (0, 0)),
            pl.BlockSpec((C, C), lambda b, t: (0, 0)),
            pl.BlockSpec((1, C), lambda b, t: (0, 0)),
            pl.BlockSpec((1, C), lambda b, t: (0, 0)),
        ],
        out_specs=pl.BlockSpec((1, 1, HP, C), lambda b, t: (b, t, 0, 0)),
        out_shape=jax.ShapeDtypeStruct((B, T, HP, C), f32),
        scratch_shapes=[pltpu.VMEM((3, SROWS, DIM), f32)],
        compiler_params=pltpu.CompilerParams(
            dimension_semantics=("parallel", "arbitrary")),
    )(at, v_f, v_f, v_f, lwt, ow_t, lb2, ob2)

    return out_f.reshape(B, T, H, W, C).transpose(0, 4, 1, 2, 3)


# speedup vs baseline: 1.9319x; 1.2337x over previous
import jax
import jax.numpy as jnp
from jax.experimental import pallas as pl
import jax.experimental.pallas.tpu as pltpu

DIM = 96
NH = 8
HD = DIM // NH
TOPK = 4
THETA = 0.2
SCALE = DIM ** (-0.5)
EPS = 1e-5

H = W = 56
HP = H * W
PADR = 64
SROWS = PADR + HP + PADR
RS = 196
RSP = 208
NREG_HW = 16
CQK = 224

f32 = jnp.float32
bf16 = jnp.bfloat16


def _wcol_masks():
    wcol = jax.lax.broadcasted_iota(jnp.int32, (HP, 1), 0) % W
    m_m1 = (wcol >= 1).astype(f32)
    m_p1 = (wcol <= W - 2).astype(f32)
    return m_m1, m_p1


def _slab_to_regions(flat, c):
    chunks = []
    for rh in range(4):
        blk = flat[rh * 784:(rh + 1) * 784, :].reshape(14, 4, 14, c)
        chunks.append(blk.transpose(1, 0, 2, 3).reshape(4, RS, c))
    return jnp.concatenate(chunks, axis=0)


def _conv_kernel(xp_ref, xc_ref, xn_ref, wtap_ref, kd_ref, wv_ref,
                 q_ref, k_ref, vr_ref, vf_ref, pool_ref, scr):
    t = pl.program_id(1)
    T = pl.num_programs(1)
    scr[:, :PADR, :] = jnp.zeros((3, PADR, DIM), f32)
    scr[:, PADR + HP:, :] = jnp.zeros((3, PADR, DIM), f32)
    mp = (t > 0).astype(f32)
    mn = (t < T - 1).astype(f32)
    scr[0, PADR:PADR + HP, :] = xp_ref[0, 0] * mp
    scr[1, PADR:PADR + HP, :] = xc_ref[0, 0]
    scr[2, PADR:PADR + HP, :] = xn_ref[0, 0] * mn

    m_m1, m_p1 = _wcol_masks()
    wmask = {-1: m_m1, 1: m_p1}

    acc = jnp.zeros((HP, CQK), f32)
    i = 0
    for dt in range(3):
        for a in (-1, 0, 1):
            for b in (-1, 0, 1):
                off = PADR + a * W + b
                win = scr[dt, off:off + HP, :]
                if b != 0:
                    win = win * wmask[b]
                acc = acc + jnp.dot(win.astype(bf16), wtap_ref[i],
                                    preferred_element_type=f32)
                i += 1

    xc16 = xc_ref[0, 0].astype(bf16)
    acc = acc - THETA * jnp.dot(xc16, kd_ref[...], preferred_element_type=f32)

    v = jnp.dot(xc16, wv_ref[...], preferred_element_type=f32)

    ir = jax.lax.broadcasted_iota(jnp.int32, (NREG_HW + 1, HP), 0)
    p = jax.lax.broadcasted_iota(jnp.int32, (NREG_HW + 1, HP), 1)
    reg = (p // (W * 14)) * 4 + (p % W) // 14
    pmat = ((ir == reg) | (ir == NREG_HW)).astype(bf16)
    acc_hi = acc.astype(bf16)
    acc_lo = (acc - acc_hi.astype(f32)).astype(bf16)
    nt = (((1,), (0,)), ((), ()))
    pooled = (jax.lax.dot_general(pmat, acc_hi, nt, preferred_element_type=f32)
              + jax.lax.dot_general(pmat, acc_lo, nt, preferred_element_type=f32))
    sq = jnp.sum(acc * acc, axis=0).reshape(1, CQK)

    q_ref[0, :, :RS, :] = _slab_to_regions(acc[:, :DIM], DIM)
    k_ref[0, :, :RS, :] = _slab_to_regions(acc[:, 128:128 + DIM], DIM)
    vr_ref[0, :, :RS, :] = _slab_to_regions(v, DIM).astype(bf16)
    q_ref[0, :, RS:, :] = jnp.zeros((NREG_HW, RSP - RS, DIM), f32)
    k_ref[0, :, RS:, :] = jnp.zeros((NREG_HW, RSP - RS, DIM), f32)
    vr_ref[0, :, RS:, :] = jnp.zeros((NREG_HW, RSP - RS, DIM), bf16)
    vf_ref[0, 0] = v.astype(bf16)
    pool_ref[0, 0, :NREG_HW + 1, :] = pooled
    pool_ref[0, 0, NREG_HW + 1:, :] = sq


def _topk_kernel(rs_ref, scale_ref, shift_ref, idx_ref):
    rs = rs_ref[0] * (1.0 / RS)
    qr = rs[:, :DIM] * scale_ref[0, :DIM] + shift_ref[0, :DIM]
    kr = (rs[:, 128:128 + DIM] * scale_ref[0, 128:128 + DIM]
          + shift_ref[0, 128:128 + DIM])
    a = jax.lax.dot_general(qr.astype(bf16), kr.astype(bf16),
                            (((1,), (1,)), ((), ())),
                            preferred_element_type=f32)
    n = a.shape[0]
    iota = jax.lax.broadcasted_iota(jnp.int32, (n, n), 1)
    cols = []
    for _ in range(TOPK):
        m = jnp.max(a, axis=1, keepdims=True)
        col = jnp.min(jnp.where(a >= m, iota, n), axis=1)
        cols.append(col.reshape(n, 1))
        a = jnp.where(iota == col[:, None], -jnp.inf, a)
    idx_ref[0] = jnp.concatenate(cols, axis=1)


def _attn_kernel(idx_ref, q_ref, k0_ref, k1_ref, k2_ref, k3_ref,
                 v0_ref, v1_ref, v2_ref, v3_ref,
                 qsc_ref, qsh_ref, ksc_ref, ksh_ref, o_ref):
    del idx_ref
    qb = (q_ref[0, 0] * qsc_ref[0] + qsh_ref[0]) * SCALE
    kb = jnp.concatenate([k0_ref[0, 0], k1_ref[0, 0],
                          k2_ref[0, 0], k3_ref[0, 0]], axis=0)
    kb = kb * ksc_ref[0] + ksh_ref[0]
    v16 = jnp.concatenate([v0_ref[0, 0], v1_ref[0, 0],
                           v2_ref[0, 0], v3_ref[0, 0]], axis=0)
    q16, k16 = qb.astype(bf16), kb.astype(bf16)
    nk = TOPK * RSP
    kmask = (jax.lax.broadcasted_iota(jnp.int32, (1, nk), 1) % RSP) < RS
    outs = []
    for h in range(NH):
        sl = slice(h * HD, (h + 1) * HD)
        s = jax.lax.dot_general(q16[:, sl], k16[:, sl],
                                (((1,), (1,)), ((), ())),
                                preferred_element_type=f32)
        s = jnp.where(kmask, s, -jnp.inf)
        m = jnp.max(s, axis=1, keepdims=True)
        e = jnp.exp(s - m)
        p = e / jnp.sum(e, axis=1, keepdims=True)
        outs.append(jnp.dot(p.astype(bf16), v16[:, sl],
                            preferred_element_type=f32))
    o_ref[0, 0] = jnp.concatenate(outs, axis=1)


def _lepe_kernel(at_ref, vp_ref, vc_ref, vn_ref, lwt_ref, owt_ref,
                 lb_ref, ob_ref, o_ref, scr, sacc):
    t = pl.program_id(1)
    T = pl.num_programs(1)
    scr[:, :PADR, :] = jnp.zeros((3, PADR, DIM), f32)
    scr[:, PADR + HP:, :] = jnp.zeros((3, PADR, DIM), f32)
    mp = (t > 0).astype(f32)
    mn = (t < T - 1).astype(f32)
    scr[0, PADR:PADR + HP, :] = vp_ref[0, 0].astype(f32) * mp
    scr[1, PADR:PADR + HP, :] = vc_ref[0, 0].astype(f32)
    scr[2, PADR:PADR + HP, :] = vn_ref[0, 0].astype(f32) * mn

    lw = lwt_ref[...]
    ext = HP + 16
    for bi, b in enumerate((-1, 0, 1)):
        ab = jnp.zeros((ext, DIM), f32)
        for dt in range(3):
            for ai, a in enumerate((-1, 0, 1)):
                i = dt * 9 + ai * 3 + (b + 1)
                off = PADR - 8 + a * W
                ab = ab + scr[dt, off:off + ext, :] * lw[i].reshape(1, DIM)
        sacc[bi] = ab

    m_m1, m_p1 = _wcol_masks()
    lepe = (sacc[0, 8 - 1:8 - 1 + HP, :] * m_m1
            + sacc[1, 8:8 + HP, :]
            + sacc[2, 8 + 1:8 + 1 + HP, :] * m_p1)

    chunks = []
    for rh in range(4):
        blk = at_ref[0, 0, rh * 4:(rh + 1) * 4, :RS, :]
        blk = blk.reshape(4, 14, 14, DIM).transpose(1, 0, 2, 3)
        chunks.append(blk.reshape(784, DIM))
    attn = jnp.concatenate(chunks, axis=0)

    total = attn + lepe + lb_ref[0]
    out = jnp.dot(total.astype(bf16), owt_ref[...],
                  preferred_element_type=f32) + ob_ref[0]
    o_ref[0, 0] = out


def _tap_weights(w, col):
    o, i = w.shape[0], w.shape[1]
    wt = w.transpose(2, 3, 4, 1, 0).reshape(27, i, o)
    return jnp.zeros((27, i, CQK), w.dtype).at[:, :, col:col + o].set(wt)


def _cdc_diff_weight(w, col):
    kd = w[:, :, 0].sum(axis=(2, 3)) + w[:, :, 2].sum(axis=(2, 3))
    i, o = kd.shape[1], kd.shape[0]
    return jnp.zeros((i, CQK), w.dtype).at[:, col:col + o].set(kd.T)


@jax.jit
def kernel(x, wq, gq, bq, wk, gk, bk, wv, lw, lb, ow, ob):
    B, C, T, _, _ = x.shape
    R = T * NREG_HW
    N = B * T * H * W

    xf = x.transpose(0, 2, 3, 4, 1).reshape(B, T, HP, C)
    wqk = (_tap_weights(wq, 0) + _tap_weights(wk, 128)).astype(bf16)
    kd2 = (_cdc_diff_weight(wq, 0) + _cdc_diff_weight(wk, 128)).astype(bf16)
    wv_t = wv.T.astype(bf16)

    grid1 = (B, T)
    qreg, kreg, vreg, v_f, pools = pl.pallas_call(
        _conv_kernel,
        grid=grid1,
        in_specs=[
            pl.BlockSpec((1, 1, HP, C),
                         lambda b, t: (b, jnp.maximum(t - 1, 0), 0, 0)),
            pl.BlockSpec((1, 1, HP, C), lambda b, t: (b, t, 0, 0)),
            pl.BlockSpec((1, 1, HP, C),
                         lambda b, t: (b, jnp.minimum(t + 1, T - 1), 0, 0)),
            pl.BlockSpec((27, C, CQK), lambda b, t: (0, 0, 0)),
            pl.BlockSpec((C, CQK), lambda b, t: (0, 0)),
            pl.BlockSpec((C, C), lambda b, t: (0, 0)),
        ],
        out_specs=[
            pl.BlockSpec((1, NREG_HW, RSP, C), lambda b, t: (b, t, 0, 0)),
            pl.BlockSpec((1, NREG_HW, RSP, C), lambda b, t: (b, t, 0, 0)),
            pl.BlockSpec((1, NREG_HW, RSP, C), lambda b, t: (b, t, 0, 0)),
            pl.BlockSpec((1, 1, HP, C), lambda b, t: (b, t, 0, 0)),
            pl.BlockSpec((1, 1, NREG_HW + 2, CQK),
                         lambda b, t: (b, t, 0, 0)),
        ],
        out_shape=[
            jax.ShapeDtypeStruct((B, R, RSP, C), f32),
            jax.ShapeDtypeStruct((B, R, RSP, C), f32),
            jax.ShapeDtypeStruct((B, R, RSP, C), bf16),
            jax.ShapeDtypeStruct((B, T, HP, C), bf16),
            jax.ShapeDtypeStruct((B, T, NREG_HW + 2, CQK), f32),
        ],
        scratch_shapes=[pltpu.VMEM((3, SROWS, DIM), f32)],
        compiler_params=pltpu.CompilerParams(
            dimension_semantics=("parallel", "arbitrary")),
    )(xf, xf, xf, wqk, kd2, wv_t)

    ssum = pools[:, :, NREG_HW, :].sum(axis=(0, 1))
    ssq = pools[:, :, NREG_HW + 1, :].sum(axis=(0, 1))
    mean = ssum / N
    var = ssq / N - mean * mean
    g2 = jnp.zeros((CQK,), f32).at[:C].set(gq).at[128:128 + C].set(gk)
    b2 = jnp.zeros((CQK,), f32).at[:C].set(bq).at[128:128 + C].set(bk)
    scale = g2 / jnp.sqrt(var + EPS)
    shift = b2 - mean * scale
    scale2 = scale.reshape(1, CQK)
    shift2 = shift.reshape(1, CQK)

    rsums = pools[:, :, :NREG_HW, :].reshape(B, R, CQK)
    idx = pl.pallas_call(
        _topk_kernel,
        grid=(B,),
        in_specs=[
            pl.BlockSpec((1, R, CQK), lambda b: (b, 0, 0)),
            pl.BlockSpec((1, CQK), lambda b: (0, 0)),
            pl.BlockSpec((1, CQK), lambda b: (0, 0)),
        ],
        out_specs=pl.BlockSpec((1, R, TOPK), lambda b: (b, 0, 0)),
        out_shape=jax.ShapeDtypeStruct((B, R, TOPK), jnp.int32),
    )(rsums, scale2, shift2)

    qsc, qsh = scale2[:, :C], shift2[:, :C]
    ksc, ksh = scale2[:, 128:128 + C], shift2[:, 128:128 + C]

    def qmap(b, r, idx_ref):
        return (b, r, 0, 0)

    def gmap(j):
        def m(b, r, idx_ref):
            return (b, idx_ref[b, r, j], 0, 0)
        return m

    def vecmap(b, r, idx_ref):
        return (0, 0)

    gspec = pltpu.PrefetchScalarGridSpec(
        num_scalar_prefetch=1,
        grid=(B, R),
        in_specs=[
            pl.BlockSpec((1, 1, RSP, C), qmap),
            pl.BlockSpec((1, 1, RSP, C), gmap(0)),
            pl.BlockSpec((1, 1, RSP, C), gmap(1)),
            pl.BlockSpec((1, 1, RSP, C), gmap(2)),
            pl.BlockSpec((1, 1, RSP, C), gmap(3)),
            pl.BlockSpec((1, 1, RSP, C), gmap(0)),
            pl.BlockSpec((1, 1, RSP, C), gmap(1)),
            pl.BlockSpec((1, 1, RSP, C), gmap(2)),
            pl.BlockSpec((1, 1, RSP, C), gmap(3)),
            pl.BlockSpec((1, C), vecmap),
            pl.BlockSpec((1, C), vecmap),
            pl.BlockSpec((1, C), vecmap),
            pl.BlockSpec((1, C), vecmap),
        ],
        out_specs=pl.BlockSpec((1, 1, RSP, C), qmap),
    )
    attn_out = pl.pallas_call(
        _attn_kernel,
        grid_spec=gspec,
        out_shape=jax.ShapeDtypeStruct((B, R, RSP, C), f32),
        compiler_params=pltpu.CompilerParams(
            dimension_semantics=("parallel", "arbitrary")),
    )(idx, qreg, kreg, kreg, kreg, kreg, vreg, vreg, vreg, vreg,
      qsc, qsh, ksc, ksh)

    lwt = lw[:, 0].transpose(1, 2, 3, 0).reshape(27, C).astype(bf16).astype(f32)
    ow_t = ow.T.astype(bf16)
    lb2 = lb.reshape(1, C)
    ob2 = ob.reshape(1, C)

    at4 = attn_out.reshape(B, T, NREG_HW, RSP, C)
    out_f = pl.pallas_call(
        _lepe_kernel,
        grid=grid1,
        in_specs=[
            pl.BlockSpec((1, 1, NREG_HW, RSP, C),
                         lambda b, t: (b, t, 0, 0, 0)),
            pl.BlockSpec((1, 1, HP, C),
                         lambda b, t: (b, jnp.maximum(t - 1, 0), 0, 0)),
            pl.BlockSpec((1, 1, HP, C), lambda b, t: (b, t, 0, 0)),
            pl.BlockSpec((1, 1, HP, C),
                         lambda b, t: (b, jnp.minimum(t + 1, T - 1), 0, 0)),
            pl.BlockSpec((27, C), lambda b, t: (0, 0)),
            pl.BlockSpec((C, C), lambda b, t: (0, 0)),
            pl.BlockSpec((1, C), lambda b, t: (0, 0)),
            pl.BlockSpec((1, C), lambda b, t: (0, 0)),
        ],
        out_specs=pl.BlockSpec((1, 1, HP, C), lambda b, t: (b, t, 0, 0)),
        out_shape=jax.ShapeDtypeStruct((B, T, HP, C), f32),
        scratch_shapes=[pltpu.VMEM((3, SROWS, DIM), f32),
                        pltpu.VMEM((3, HP + 16, DIM), f32)],
        compiler_params=pltpu.CompilerParams(
            dimension_semantics=("parallel", "arbitrary")),
    )(at4, v_f, v_f, v_f, lwt, ow_t, lb2, ob2)

    return out_f.reshape(B, T, H, W, C).transpose(0, 4, 1, 2, 3)


# speedup vs baseline: 1.9688x; 1.0191x over previous
import jax
import jax.numpy as jnp
from jax.experimental import pallas as pl
import jax.experimental.pallas.tpu as pltpu

DIM = 96
NH = 8
HD = DIM // NH
TOPK = 4
THETA = 0.2
SCALE = DIM ** (-0.5)
EPS = 1e-5

H = W = 56
HP = H * W
PADR = 64
SROWS = PADR + HP + PADR
RS = 196
RSP = 208
NREG_HW = 16
CQK = 224

f32 = jnp.float32
bf16 = jnp.bfloat16


def _wcol_masks():
    wcol = jax.lax.broadcasted_iota(jnp.int32, (HP, 1), 0) % W
    m_m1 = (wcol >= 1).astype(f32)
    m_p1 = (wcol <= W - 2).astype(f32)
    return m_m1, m_p1


def _slab_to_regions(flat, c):
    chunks = []
    for rh in range(4):
        blk = flat[rh * 784:(rh + 1) * 784, :].reshape(14, 4, 14, c)
        chunks.append(blk.transpose(1, 0, 2, 3).reshape(4, RS, c))
    return jnp.concatenate(chunks, axis=0)


def _conv_kernel(xp_ref, xc_ref, xn_ref, wtap_ref, kd_ref, wv_ref,
                 q_ref, k_ref, vr_ref, vf_ref, pool_ref, scr):
    t = pl.program_id(1)
    T = pl.num_programs(1)
    scr[:, :PADR, :] = jnp.zeros((3, PADR, DIM), f32)
    scr[:, PADR + HP:, :] = jnp.zeros((3, PADR, DIM), f32)
    mp = (t > 0).astype(f32)
    mn = (t < T - 1).astype(f32)
    scr[0, PADR:PADR + HP, :] = xp_ref[0, 0] * mp
    scr[1, PADR:PADR + HP, :] = xc_ref[0, 0]
    scr[2, PADR:PADR + HP, :] = xn_ref[0, 0] * mn

    m_m1, m_p1 = _wcol_masks()
    wmask = {-1: m_m1, 1: m_p1}

    acc = jnp.zeros((HP, CQK), f32)
    i = 0
    for dt in range(3):
        for a in (-1, 0, 1):
            for b in (-1, 0, 1):
                off = PADR + a * W + b
                win = scr[dt, off:off + HP, :]
                if b != 0:
                    win = win * wmask[b]
                acc = acc + jnp.dot(win.astype(bf16), wtap_ref[i],
                                    preferred_element_type=f32)
                i += 1

    xc16 = xc_ref[0, 0].astype(bf16)
    acc = acc - THETA * jnp.dot(xc16, kd_ref[...], preferred_element_type=f32)

    v = jnp.dot(xc16, wv_ref[...], preferred_element_type=f32)

    ir = jax.lax.broadcasted_iota(jnp.int32, (NREG_HW + 1, HP), 0)
    p = jax.lax.broadcasted_iota(jnp.int32, (NREG_HW + 1, HP), 1)
    reg = (p // (W * 14)) * 4 + (p % W) // 14
    pmat = ((ir == reg) | (ir == NREG_HW)).astype(bf16)
    acc_hi = acc.astype(bf16)
    acc_lo = (acc - acc_hi.astype(f32)).astype(bf16)
    nt = (((1,), (0,)), ((), ()))
    pooled = (jax.lax.dot_general(pmat, acc_hi, nt, preferred_element_type=f32)
              + jax.lax.dot_general(pmat, acc_lo, nt, preferred_element_type=f32))
    sq = jnp.sum(acc * acc, axis=0).reshape(1, CQK)

    q_ref[0, :, :RS, :] = _slab_to_regions(acc[:, :DIM], DIM)
    k_ref[0, :, :RS, :] = _slab_to_regions(acc[:, 128:128 + DIM], DIM)
    vr_ref[0, :, :RS, :] = _slab_to_regions(v, DIM).astype(bf16)
    q_ref[0, :, RS:, :] = jnp.zeros((NREG_HW, RSP - RS, DIM), f32)
    k_ref[0, :, RS:, :] = jnp.zeros((NREG_HW, RSP - RS, DIM), f32)
    vr_ref[0, :, RS:, :] = jnp.zeros((NREG_HW, RSP - RS, DIM), bf16)
    vf_ref[0, 0] = v.astype(bf16)
    pool_ref[0, 0, :NREG_HW + 1, :] = pooled
    pool_ref[0, 0, NREG_HW + 1:, :] = sq


def _topk_kernel(rs_ref, scale_ref, shift_ref, idx_ref):
    rs = rs_ref[0] * (1.0 / RS)
    qr = rs[:, :DIM] * scale_ref[0, :DIM] + shift_ref[0, :DIM]
    kr = (rs[:, 128:128 + DIM] * scale_ref[0, 128:128 + DIM]
          + shift_ref[0, 128:128 + DIM])
    a = jax.lax.dot_general(qr.astype(bf16), kr.astype(bf16),
                            (((1,), (1,)), ((), ())),
                            preferred_element_type=f32)
    n = a.shape[0]
    iota = jax.lax.broadcasted_iota(jnp.int32, (n, n), 1)
    cols = []
    for _ in range(TOPK):
        m = jnp.max(a, axis=1, keepdims=True)
        col = jnp.min(jnp.where(a >= m, iota, n), axis=1)
        cols.append(col.reshape(n, 1))
        a = jnp.where(iota == col[:, None], -jnp.inf, a)
    idx_ref[0] = jnp.concatenate(cols, axis=1)


def _attn_kernel(idx_ref, q_ref,
                 k00_ref, k01_ref, k02_ref, k03_ref,
                 k10_ref, k11_ref, k12_ref, k13_ref,
                 v00_ref, v01_ref, v02_ref, v03_ref,
                 v10_ref, v11_ref, v12_ref, v13_ref,
                 qsc_ref, qsh_ref, ksc_ref, ksh_ref, o_ref):
    del idx_ref
    krefs = [[k00_ref, k01_ref, k02_ref, k03_ref],
             [k10_ref, k11_ref, k12_ref, k13_ref]]
    vrefs = [[v00_ref, v01_ref, v02_ref, v03_ref],
             [v10_ref, v11_ref, v12_ref, v13_ref]]
    nk = TOPK * RSP
    kmask = (jax.lax.broadcasted_iota(jnp.int32, (1, nk), 1) % RSP) < RS
    for i in range(2):
        qb = (q_ref[0, i] * qsc_ref[0] + qsh_ref[0]) * SCALE
        kb = jnp.concatenate([r[0, 0] for r in krefs[i]], axis=0)
        kb = kb * ksc_ref[0] + ksh_ref[0]
        v16 = jnp.concatenate([r[0, 0] for r in vrefs[i]], axis=0)
        q16, k16 = qb.astype(bf16), kb.astype(bf16)
        outs = []
        for h in range(NH):
            sl = slice(h * HD, (h + 1) * HD)
            s = jax.lax.dot_general(q16[:, sl], k16[:, sl],
                                    (((1,), (1,)), ((), ())),
                                    preferred_element_type=f32)
            s = jnp.where(kmask, s, -jnp.inf)
            m = jnp.max(s, axis=1, keepdims=True)
            e = jnp.exp(s - m)
            p = e / jnp.sum(e, axis=1, keepdims=True)
            outs.append(jnp.dot(p.astype(bf16), v16[:, sl],
                                preferred_element_type=f32))
        o_ref[0, i] = jnp.concatenate(outs, axis=1)


def _lepe_kernel(at_ref, vp_ref, vc_ref, vn_ref, lwt_ref, owt_ref,
                 lb_ref, ob_ref, o_ref, scr, sacc):
    t = pl.program_id(1)
    T = pl.num_programs(1)
    scr[:, :PADR, :] = jnp.zeros((3, PADR, DIM), f32)
    scr[:, PADR + HP:, :] = jnp.zeros((3, PADR, DIM), f32)
    mp = (t > 0).astype(f32)
    mn = (t < T - 1).astype(f32)
    scr[0, PADR:PADR + HP, :] = vp_ref[0, 0].astype(f32) * mp
    scr[1, PADR:PADR + HP, :] = vc_ref[0, 0].astype(f32)
    scr[2, PADR:PADR + HP, :] = vn_ref[0, 0].astype(f32) * mn

    lw = lwt_ref[...]
    ext = HP + 16
    for bi, b in enumerate((-1, 0, 1)):
        ab = jnp.zeros((ext, DIM), f32)
        for dt in range(3):
            for ai, a in enumerate((-1, 0, 1)):
                i = dt * 9 + ai * 3 + (b + 1)
                off = PADR - 8 + a * W
                ab = ab + scr[dt, off:off + ext, :] * lw[i].reshape(1, DIM)
        sacc[bi] = ab

    m_m1, m_p1 = _wcol_masks()
    lepe = (sacc[0, 8 - 1:8 - 1 + HP, :] * m_m1
            + sacc[1, 8:8 + HP, :]
            + sacc[2, 8 + 1:8 + 1 + HP, :] * m_p1)

    chunks = []
    for rh in range(4):
        blk = at_ref[0, 0, rh * 4:(rh + 1) * 4, :RS, :]
        blk = blk.reshape(4, 14, 14, DIM).transpose(1, 0, 2, 3)
        chunks.append(blk.reshape(784, DIM))
    attn = jnp.concatenate(chunks, axis=0)

    total = attn + lepe + lb_ref[0]
    out = jnp.dot(total.astype(bf16), owt_ref[...],
                  preferred_element_type=f32) + ob_ref[0]
    o_ref[0, 0] = out


def _tap_weights(w, col):
    o, i = w.shape[0], w.shape[1]
    wt = w.transpose(2, 3, 4, 1, 0).reshape(27, i, o)
    return jnp.zeros((27, i, CQK), w.dtype).at[:, :, col:col + o].set(wt)


def _cdc_diff_weight(w, col):
    kd = w[:, :, 0].sum(axis=(2, 3)) + w[:, :, 2].sum(axis=(2, 3))
    i, o = kd.shape[1], kd.shape[0]
    return jnp.zeros((i, CQK), w.dtype).at[:, col:col + o].set(kd.T)


@jax.jit
def kernel(x, wq, gq, bq, wk, gk, bk, wv, lw, lb, ow, ob):
    B, C, T, _, _ = x.shape
    R = T * NREG_HW
    N = B * T * H * W

    xf = x.transpose(0, 2, 3, 4, 1).reshape(B, T, HP, C)
    wqk = (_tap_weights(wq, 0) + _tap_weights(wk, 128)).astype(bf16)
    kd2 = (_cdc_diff_weight(wq, 0) + _cdc_diff_weight(wk, 128)).astype(bf16)
    wv_t = wv.T.astype(bf16)

    grid1 = (B, T)
    qreg, kreg, vreg, v_f, pools = pl.pallas_call(
        _conv_kernel,
        grid=grid1,
        in_specs=[
            pl.BlockSpec((1, 1, HP, C),
                         lambda b, t: (b, jnp.maximum(t - 1, 0), 0, 0)),
            pl.BlockSpec((1, 1, HP, C), lambda b, t: (b, t, 0, 0)),
            pl.BlockSpec((1, 1, HP, C),
                         lambda b, t: (b, jnp.minimum(t + 1, T - 1), 0, 0)),
            pl.BlockSpec((27, C, CQK), lambda b, t: (0, 0, 0)),
            pl.BlockSpec((C, CQK), lambda b, t: (0, 0)),
            pl.BlockSpec((C, C), lambda b, t: (0, 0)),
        ],
        out_specs=[
            pl.BlockSpec((1, NREG_HW, RSP, C), lambda b, t: (b, t, 0, 0)),
            pl.BlockSpec((1, NREG_HW, RSP, C), lambda b, t: (b, t, 0, 0)),
            pl.BlockSpec((1, NREG_HW, RSP, C), lambda b, t: (b, t, 0, 0)),
            pl.BlockSpec((1, 1, HP, C), lambda b, t: (b, t, 0, 0)),
            pl.BlockSpec((1, 1, NREG_HW + 2, CQK),
                         lambda b, t: (b, t, 0, 0)),
        ],
        out_shape=[
            jax.ShapeDtypeStruct((B, R, RSP, C), f32),
            jax.ShapeDtypeStruct((B, R, RSP, C), f32),
            jax.ShapeDtypeStruct((B, R, RSP, C), bf16),
            jax.ShapeDtypeStruct((B, T, HP, C), bf16),
            jax.ShapeDtypeStruct((B, T, NREG_HW + 2, CQK), f32),
        ],
        scratch_shapes=[pltpu.VMEM((3, SROWS, DIM), f32)],
        compiler_params=pltpu.CompilerParams(
            dimension_semantics=("parallel", "parallel")),
    )(xf, xf, xf, wqk, kd2, wv_t)

    ssum = pools[:, :, NREG_HW, :].sum(axis=(0, 1))
    ssq = pools[:, :, NREG_HW + 1, :].sum(axis=(0, 1))
    mean = ssum / N
    var = ssq / N - mean * mean
    g2 = jnp.zeros((CQK,), f32).at[:C].set(gq).at[128:128 + C].set(gk)
    b2 = jnp.zeros((CQK,), f32).at[:C].set(bq).at[128:128 + C].set(bk)
    scale = g2 / jnp.sqrt(var + EPS)
    shift = b2 - mean * scale
    scale2 = scale.reshape(1, CQK)
    shift2 = shift.reshape(1, CQK)

    rsums = pools[:, :, :NREG_HW, :].reshape(B, R, CQK)
    idx = pl.pallas_call(
        _topk_kernel,
        grid=(B,),
        in_specs=[
            pl.BlockSpec((1, R, CQK), lambda b: (b, 0, 0)),
            pl.BlockSpec((1, CQK), lambda b: (0, 0)),
            pl.BlockSpec((1, CQK), lambda b: (0, 0)),
        ],
        out_specs=pl.BlockSpec((1, R, TOPK), lambda b: (b, 0, 0)),
        out_shape=jax.ShapeDtypeStruct((B, R, TOPK), jnp.int32),
    )(rsums, scale2, shift2)

    qsc, qsh = scale2[:, :C], shift2[:, :C]
    ksc, ksh = scale2[:, 128:128 + C], shift2[:, 128:128 + C]

    def qmap(b, r, idx_ref):
        return (b, r, 0, 0)

    def gmap(i, j):
        def m(b, r, idx_ref):
            return (b, idx_ref[b, 2 * r + i, j], 0, 0)
        return m

    def vecmap(b, r, idx_ref):
        return (0, 0)

    gather_specs = [pl.BlockSpec((1, 1, RSP, C), gmap(i, j))
                    for i in range(2) for j in range(TOPK)]
    gspec = pltpu.PrefetchScalarGridSpec(
        num_scalar_prefetch=1,
        grid=(B, R // 2),
        in_specs=(
            [pl.BlockSpec((1, 2, RSP, C), qmap)]
            + gather_specs + gather_specs
            + [pl.BlockSpec((1, C), vecmap)] * 4
        ),
        out_specs=pl.BlockSpec((1, 2, RSP, C), qmap),
    )
    attn_out = pl.pallas_call(
        _attn_kernel,
        grid_spec=gspec,
        out_shape=jax.ShapeDtypeStruct((B, R, RSP, C), f32),
        compiler_params=pltpu.CompilerParams(
            dimension_semantics=("parallel", "parallel")),
    )(idx, qreg, *([kreg] * 8), *([vreg] * 8),
      qsc, qsh, ksc, ksh)

    lwt = lw[:, 0].transpose(1, 2, 3, 0).reshape(27, C).astype(bf16).astype(f32)
    ow_t = ow.T.astype(bf16)
    lb2 = lb.reshape(1, C)
    ob2 = ob.reshape(1, C)

    at4 = attn_out.reshape(B, T, NREG_HW, RSP, C)
    out_f = pl.pallas_call(
        _lepe_kernel,
        grid=grid1,
        in_specs=[
            pl.BlockSpec((1, 1, NREG_HW, RSP, C),
                         lambda b, t: (b, t, 0, 0, 0)),
            pl.BlockSpec((1, 1, HP, C),
                         lambda b, t: (b, jnp.maximum(t - 1, 0), 0, 0)),
            pl.BlockSpec((1, 1, HP, C), lambda b, t: (b, t, 0, 0)),
            pl.BlockSpec((1, 1, HP, C),
                         lambda b, t: (b, jnp.minimum(t + 1, T - 1), 0, 0)),
            pl.BlockSpec((27, C), lambda b, t: (0, 0)),
            pl.BlockSpec((C, C), lambda b, t: (0, 0)),
            pl.BlockSpec((1, C), lambda b, t: (0, 0)),
            pl.BlockSpec((1, C), lambda b, t: (0, 0)),
        ],
        out_specs=pl.BlockSpec((1, 1, HP, C), lambda b, t: (b, t, 0, 0)),
        out_shape=jax.ShapeDtypeStruct((B, T, HP, C), f32),
        scratch_shapes=[pltpu.VMEM((3, SROWS, DIM), f32),
                        pltpu.VMEM((3, HP + 16, DIM), f32)],
        compiler_params=pltpu.CompilerParams(
            dimension_semantics=("parallel", "parallel")),
    )(at4, v_f, v_f, v_f, lwt, ow_t, lb2, ob2)

    return out_f.reshape(B, T, H, W, C).transpose(0, 4, 1, 2, 3)
